# baseline (speedup 1.0000x reference)
import functools

import jax
import jax.numpy as jnp
from jax import lax
from jax.experimental import pallas as pl
from jax.experimental.pallas import tpu as pltpu
from jax.experimental.pallas import tpu_sc as plsc

F32 = jnp.float32
L = 16
NC = 2
NS = 16
NW = NC * NS
CHUNK = 32
CHUNK_D = 128


def _sc_mesh():
    return plsc.VectorSubcoreMesh(core_axis_name="c", subcore_axis_name="s")


def _zero_vmem_2d(ref, rows, cols):
    def body(i, _):
        def inner(j, _):
            ref[i, pl.ds(j * L, L)] = jnp.zeros((L,), F32)
            return 0
        return lax.fori_loop(0, cols // L, inner, 0)
    lax.fori_loop(0, rows, body, 0)


def _make_deg_kernel(nchunk, npad):
    sl = npad // NS

    @functools.partial(
        pl.kernel,
        out_type=jax.ShapeDtypeStruct((NC, npad), F32),
        mesh=_sc_mesh(),
        scratch_types=[
            pltpu.VMEM((nchunk, CHUNK_D), jnp.int32),
            pltpu.VMEM((CHUNK_D,), F32),
            pltpu.VMEM((sl,), F32),
            pltpu.VMEM_SHARED((npad,), F32),
            pltpu.SemaphoreType.DMA,
        ],
    )
    def deg_kernel(col_hbm, deg_out, col_v, ones_v, tmp_v, shared_deg, sem):
        c = lax.axis_index("c")
        s = lax.axis_index("s")
        wid = s * NC + c

        def zb(i, _):
            tmp_v[pl.ds(i * L, L)] = jnp.zeros((L,), F32)
            return 0
        lax.fori_loop(0, sl // L, zb, 0)

        def ob(i, _):
            ones_v[pl.ds(i * L, L)] = jnp.ones((L,), F32)
            return 0
        lax.fori_loop(0, CHUNK_D // L, ob, 0)

        pltpu.sync_copy(tmp_v, shared_deg.at[pl.ds(s * sl, sl)])
        plsc.subcore_barrier()

        pltpu.sync_copy(col_hbm.at[wid], col_v)

        def edge_chunk(j, _):
            pltpu.sync_copy(ones_v, shared_deg.at[col_v.at[j]], add=True)
            return 0
        lax.fori_loop(0, nchunk, edge_chunk, 0)

        plsc.subcore_barrier()
        pltpu.sync_copy(shared_deg.at[pl.ds(s * sl, sl)], tmp_v)
        pltpu.sync_copy(tmp_v, deg_out.at[c, pl.ds(s * sl, sl)])

    return deg_kernel


NB = 16
DEPTH = 8


def _make_scatter_kernel(nchunk, npad, d):
    sl = npad // NS
    kch = sl // CHUNK
    nblk = -(-nchunk // NB)

    @functools.partial(
        pl.kernel,
        out_type=jax.ShapeDtypeStruct((NC, npad, d), F32),
        mesh=_sc_mesh(),
        scratch_types=[
            pltpu.VMEM((2, NB, CHUNK), jnp.int32),
            pltpu.VMEM((2, NB, CHUNK), jnp.int32),
            pltpu.VMEM((DEPTH, CHUNK, d), F32),
            pltpu.VMEM_SHARED((npad, d), F32),
            [pltpu.SemaphoreType.DMA] * DEPTH,
            pltpu.SemaphoreType.DMA,
        ],
    )
    def scatter_kernel(hp_hbm, row_hbm, col_hbm, part_out,
                       row_v, col_v, buf, acc, gsems, isem):
        c = lax.axis_index("c")
        s = lax.axis_index("s")
        wid = s * NC + c

        _zero_vmem_2d(buf.at[0], CHUNK, d)
        for k in range(kch):
            pltpu.sync_copy(buf.at[0],
                            acc.at[pl.ds(s * sl + k * CHUNK, CHUNK)])
        plsc.subcore_barrier()

        def idx_copies(b, nb):
            slot = b % 2
            src_r = row_hbm.at[wid, pl.ds(b * NB, nb)]
            src_c = col_hbm.at[wid, pl.ds(b * NB, nb)]
            dst_r = row_v.at[slot, pl.ds(0, nb)]
            dst_c = col_v.at[slot, pl.ds(0, nb)]
            return (src_r, dst_r), (src_c, dst_c)

        (sr, dr), (sc_, dc) = idx_copies(0, min(NB, nchunk))
        pltpu.sync_copy(sr, dr)
        pltpu.sync_copy(sc_, dc)

        for b in range(nblk):
            slot = b % 2
            nb = min(NB, nchunk - b * NB)
            if b > 0:
                (sr, dr), (sc_, dc) = idx_copies(b, nb)
                pltpu.make_async_copy(sr, dr, isem).wait()
                pltpu.make_async_copy(sc_, dc, isem).wait()
            if b + 1 < nblk:
                (sr, dr), (sc_, dc) = idx_copies(b + 1,
                                                 min(NB, nchunk - (b + 1) * NB))
                pltpu.async_copy(sr, dr, isem)
                pltpu.async_copy(sc_, dc, isem)

            rv = row_v.at[slot]
            cv = col_v.at[slot]

            def gath(j, k):
                pltpu.async_copy(hp_hbm.at[rv.at[j]], buf.at[k], gsems[k])

            def gath_wait(j, k):
                pltpu.make_async_copy(hp_hbm.at[rv.at[j]], buf.at[k],
                                      gsems[k]).wait()

            for k in range(DEPTH):
                if k < nb:
                    gath(k, k)

            def quad(i, _):
                j = i * DEPTH
                for k in range(DEPTH):
                    @pl.when(j + k < nb)
                    def _(k=k):
                        gath_wait(j + k, k)
                        pltpu.sync_copy(buf.at[k], acc.at[cv.at[j + k]],
                                        add=True)

                        @pl.when(j + k + DEPTH < nb)
                        def _():
                            gath(j + k + DEPTH, k)
                return 0
            lax.fori_loop(0, -(-nb // DEPTH), quad, 0)

        plsc.subcore_barrier()
        for k in range(kch):
            base = s * sl + k * CHUNK
            pltpu.sync_copy(acc.at[pl.ds(base, CHUNK)], buf.at[0])
            pltpu.sync_copy(buf.at[0], part_out.at[c, pl.ds(base, CHUNK)])

    return scatter_kernel


def _dinv_of(deg2_blk):
    return lax.rsqrt(deg2_blk[:, 0] + deg2_blk[:, 1] + 1.0)


def _hprime_body(x_ref, w_ref, deg2_ref, out_ref):
    dinv = _dinv_of(deg2_ref[...])
    h = jnp.dot(x_ref[...], w_ref[...], preferred_element_type=F32)
    out_ref[...] = h * dinv[:, None]


def _hprime(x, w, deg2, br):
    n, d = x.shape
    grid = n // br
    return pl.pallas_call(
        _hprime_body,
        grid=(grid,),
        in_specs=[
            pl.BlockSpec((br, d), lambda i: (i, 0)),
            pl.BlockSpec((d, d), lambda i: (0, 0)),
            pl.BlockSpec((br, 2), lambda i: (i, 0)),
        ],
        out_specs=pl.BlockSpec((br, d), lambda i: (i, 0)),
        out_shape=jax.ShapeDtypeStruct((n, d), F32),
    )(x, w, deg2)


def _stage_body(n, br, relu, matmul, part_ref, hp_ref, deg2_ref, g_ref,
                be_ref, w_ref, out_ref, c_s, sums_s):
    ph = pl.program_id(0)
    i = pl.program_id(1)
    dinv = _dinv_of(deg2_ref[...])

    @pl.when(ph == 0)
    def _():
        cc = (part_ref[0] + part_ref[1] + hp_ref[...]) * dinv[:, None]
        c_s[pl.ds(i * br, br), :] = cc

        @pl.when(i == 0)
        def _():
            sums_s[...] = jnp.zeros_like(sums_s)
        sums_s[...] += jnp.stack([cc.sum(axis=0), (cc * cc).sum(axis=0)])

    @pl.when(ph == 1)
    def _():
        mu = sums_s[0] / n
        var = sums_s[1] / n - mu * mu
        istd = lax.rsqrt(var + 1e-5)
        y = (c_s[pl.ds(i * br, br), :] - mu) * (istd * g_ref[0]) + be_ref[0]
        if relu:
            y = jnp.maximum(y, 0.0)
        if matmul:
            y = (jnp.dot(y, w_ref[...], preferred_element_type=F32)
                 * dinv[:, None])
        out_ref[...] = y


def _stage(part, hp, deg2, g, be, w, br, relu, matmul):
    n, d = hp.shape
    grid = n // br
    return pl.pallas_call(
        functools.partial(_stage_body, float(n), br, relu, matmul),
        grid=(2, grid),
        in_specs=[
            pl.BlockSpec((NC, br, d), lambda ph, i: (0, i * (1 - ph), 0)),
            pl.BlockSpec((br, d), lambda ph, i: (i * (1 - ph), 0)),
            pl.BlockSpec((br, 2), lambda ph, i: (i, 0)),
            pl.BlockSpec((1, d), lambda ph, i: (0, 0)),
            pl.BlockSpec((1, d), lambda ph, i: (0, 0)),
            pl.BlockSpec((d, d), lambda ph, i: (0, 0)),
        ],
        out_specs=pl.BlockSpec((br, d), lambda ph, i: (i, 0)),
        out_shape=jax.ShapeDtypeStruct((n, d), F32),
        scratch_shapes=[
            pltpu.VMEM((n, d), F32),
            pltpu.VMEM((2, d), F32),
        ],
    )(part, hp, deg2, g.reshape(1, d), be.reshape(1, d), w)


def kernel(x, edge_index, edge_attr, batch, W0, b0, g0, be0, W1, b1, g1, be1):
    n, d = x.shape
    e = edge_index.shape[1]

    align = NW * CHUNK_D
    epad = -(-e // align) * align
    nchunk = epad // (NW * CHUNK)
    nchunk_d = epad // (NW * CHUNK_D)
    npad = (n // (NS * CHUNK) + 1) * (NS * CHUNK)
    br = 2000 if n % 2000 == 0 else 8 * (n // 8)

    row = edge_index[0].astype(jnp.int32)
    col = edge_index[1].astype(jnp.int32)
    nex = epad - e
    pad_src = (jnp.arange(nex, dtype=jnp.int32) * 97) % n
    pad_dst = n + jnp.arange(nex, dtype=jnp.int32) % (npad - n)
    col_f = jnp.concatenate([col, pad_dst])
    row_t = jnp.concatenate([row, pad_src]).reshape(NW, nchunk, CHUNK)
    col_t = col_f.reshape(NW, nchunk, CHUNK)
    col_d = col_f.reshape(NW, nchunk_d, CHUNK_D)

    deg_part = _make_deg_kernel(nchunk_d, npad)(col_d)
    deg2 = deg_part.T

    scatter = _make_scatter_kernel(nchunk, npad, d)

    h = x.astype(F32)
    hp0 = _hprime(h, W0, deg2, br)
    part0 = scatter(hp0, row_t, col_t)
    hp1 = _stage(part0, hp0, deg2, g0, be0, W1, br, relu=True, matmul=True)
    part1 = scatter(hp1, row_t, col_t)
    out = _stage(part1, hp1, deg2, g1, be1, W1, br, relu=False,
                 matmul=False)
    return out


# speedup vs baseline: 1.1902x; 1.1902x over previous
import functools

import jax
import jax.numpy as jnp
from jax import lax
from jax.experimental import pallas as pl
from jax.experimental.pallas import tpu as pltpu
from jax.experimental.pallas import tpu_sc as plsc

F32 = jnp.float32
L = 16
NC = 2
NS = 16
NW = NC * NS
CHUNK = 64
CHUNK_D = 128


def _sc_mesh():
    return plsc.VectorSubcoreMesh(core_axis_name="c", subcore_axis_name="s")


def _zero_vmem_2d(ref, rows, cols):
    def body(i, _):
        def inner(j, _):
            ref[i, pl.ds(j * L, L)] = jnp.zeros((L,), F32)
            return 0
        return lax.fori_loop(0, cols // L, inner, 0)
    lax.fori_loop(0, rows, body, 0)


def _make_deg_kernel(nchunk, npad):
    sl = npad // NS

    @functools.partial(
        pl.kernel,
        out_type=jax.ShapeDtypeStruct((NC, npad), F32),
        mesh=_sc_mesh(),
        scratch_types=[
            pltpu.VMEM((nchunk, CHUNK_D), jnp.int32),
            pltpu.VMEM((CHUNK_D,), F32),
            pltpu.VMEM((sl,), F32),
            pltpu.VMEM_SHARED((npad,), F32),
            pltpu.SemaphoreType.DMA,
        ],
    )
    def deg_kernel(col_hbm, deg_out, col_v, ones_v, tmp_v, shared_deg, sem):
        c = lax.axis_index("c")
        s = lax.axis_index("s")
        wid = s * NC + c

        def zb(i, _):
            tmp_v[pl.ds(i * L, L)] = jnp.zeros((L,), F32)
            return 0
        lax.fori_loop(0, sl // L, zb, 0)

        def ob(i, _):
            ones_v[pl.ds(i * L, L)] = jnp.ones((L,), F32)
            return 0
        lax.fori_loop(0, CHUNK_D // L, ob, 0)

        pltpu.sync_copy(tmp_v, shared_deg.at[pl.ds(s * sl, sl)])
        plsc.subcore_barrier()

        pltpu.sync_copy(col_hbm.at[wid], col_v)

        def edge_chunk(j, _):
            pltpu.sync_copy(ones_v, shared_deg.at[col_v.at[j]], add=True)
            return 0
        lax.fori_loop(0, nchunk, edge_chunk, 0)

        plsc.subcore_barrier()
        pltpu.sync_copy(shared_deg.at[pl.ds(s * sl, sl)], tmp_v)
        pltpu.sync_copy(tmp_v, deg_out.at[c, pl.ds(s * sl, sl)])

    return deg_kernel


NB = 32
DEPTH = 4


def _make_scatter_kernel(nchunk, npad, d):
    sl = npad // NS
    kch = sl // CHUNK
    nblk = -(-nchunk // NB)

    @functools.partial(
        pl.kernel,
        out_type=jax.ShapeDtypeStruct((NC, npad, d), F32),
        mesh=_sc_mesh(),
        scratch_types=[
            pltpu.VMEM((2, NB, CHUNK), jnp.int32),
            pltpu.VMEM((2, NB, CHUNK), jnp.int32),
            pltpu.VMEM((DEPTH, CHUNK, d), F32),
            pltpu.VMEM_SHARED((npad, d), F32),
            [pltpu.SemaphoreType.DMA] * DEPTH,
            pltpu.SemaphoreType.DMA,
        ],
    )
    def scatter_kernel(hp_hbm, row_hbm, col_hbm, part_out,
                       row_v, col_v, buf, acc, gsems, isem):
        c = lax.axis_index("c")
        s = lax.axis_index("s")
        wid = s * NC + c

        _zero_vmem_2d(buf.at[0], CHUNK, d)
        for k in range(kch):
            pltpu.sync_copy(buf.at[0],
                            acc.at[pl.ds(s * sl + k * CHUNK, CHUNK)])
        plsc.subcore_barrier()

        def idx_copies(b, nb):
            slot = b % 2
            src_r = row_hbm.at[wid, pl.ds(b * NB, nb)]
            src_c = col_hbm.at[wid, pl.ds(b * NB, nb)]
            dst_r = row_v.at[slot, pl.ds(0, nb)]
            dst_c = col_v.at[slot, pl.ds(0, nb)]
            return (src_r, dst_r), (src_c, dst_c)

        (sr, dr), (sc_, dc) = idx_copies(0, min(NB, nchunk))
        pltpu.sync_copy(sr, dr)
        pltpu.sync_copy(sc_, dc)

        for b in range(nblk):
            slot = b % 2
            nb = min(NB, nchunk - b * NB)
            if b > 0:
                (sr, dr), (sc_, dc) = idx_copies(b, nb)
                pltpu.make_async_copy(sr, dr, isem).wait()
                pltpu.make_async_copy(sc_, dc, isem).wait()
            if b + 1 < nblk:
                (sr, dr), (sc_, dc) = idx_copies(b + 1,
                                                 min(NB, nchunk - (b + 1) * NB))
                pltpu.async_copy(sr, dr, isem)
                pltpu.async_copy(sc_, dc, isem)

            rv = row_v.at[slot]
            cv = col_v.at[slot]

            def gath(j, k):
                pltpu.async_copy(hp_hbm.at[rv.at[j]], buf.at[k], gsems[k])

            def gath_wait(j, k):
                pltpu.make_async_copy(hp_hbm.at[rv.at[j]], buf.at[k],
                                      gsems[k]).wait()

            for k in range(DEPTH):
                if k < nb:
                    gath(k, k)

            def quad(i, _):
                j = i * DEPTH
                for k in range(DEPTH):
                    @pl.when(j + k < nb)
                    def _(k=k):
                        gath_wait(j + k, k)
                        pltpu.sync_copy(buf.at[k], acc.at[cv.at[j + k]],
                                        add=True)

                        @pl.when(j + k + DEPTH < nb)
                        def _():
                            gath(j + k + DEPTH, k)
                return 0
            lax.fori_loop(0, -(-nb // DEPTH), quad, 0)

        plsc.subcore_barrier()
        for k in range(kch):
            base = s * sl + k * CHUNK
            pltpu.sync_copy(acc.at[pl.ds(base, CHUNK)], buf.at[0])
            pltpu.sync_copy(buf.at[0], part_out.at[c, pl.ds(base, CHUNK)])

    return scatter_kernel


def _dinv_of(deg2_blk):
    return lax.rsqrt(deg2_blk[:, 0] + deg2_blk[:, 1] + 1.0)


def _hprime_body(x_ref, w_ref, deg2_ref, out_ref):
    dinv = _dinv_of(deg2_ref[...])
    h = jnp.dot(x_ref[...], w_ref[...], preferred_element_type=F32)
    out_ref[...] = h * dinv[:, None]


def _hprime(x, w, deg2, br):
    n, d = x.shape
    grid = n // br
    return pl.pallas_call(
        _hprime_body,
        grid=(grid,),
        in_specs=[
            pl.BlockSpec((br, d), lambda i: (i, 0)),
            pl.BlockSpec((d, d), lambda i: (0, 0)),
            pl.BlockSpec((br, 2), lambda i: (i, 0)),
        ],
        out_specs=pl.BlockSpec((br, d), lambda i: (i, 0)),
        out_shape=jax.ShapeDtypeStruct((n, d), F32),
    )(x, w, deg2)


def _stage_body(n, br, relu, matmul, part_ref, hp_ref, deg2_ref, g_ref,
                be_ref, w_ref, out_ref, c_s, sums_s):
    ph = pl.program_id(0)
    i = pl.program_id(1)
    dinv = _dinv_of(deg2_ref[...])

    @pl.when(ph == 0)
    def _():
        cc = (part_ref[0] + part_ref[1] + hp_ref[...]) * dinv[:, None]
        c_s[pl.ds(i * br, br), :] = cc

        @pl.when(i == 0)
        def _():
            sums_s[...] = jnp.zeros_like(sums_s)
        sums_s[...] += jnp.stack([cc.sum(axis=0), (cc * cc).sum(axis=0)])

    @pl.when(ph == 1)
    def _():
        mu = sums_s[0] / n
        var = sums_s[1] / n - mu * mu
        istd = lax.rsqrt(var + 1e-5)
        y = (c_s[pl.ds(i * br, br), :] - mu) * (istd * g_ref[0]) + be_ref[0]
        if relu:
            y = jnp.maximum(y, 0.0)
        if matmul:
            y = (jnp.dot(y, w_ref[...], preferred_element_type=F32)
                 * dinv[:, None])
        out_ref[...] = y


def _stage(part, hp, deg2, g, be, w, br, relu, matmul):
    n, d = hp.shape
    grid = n // br
    return pl.pallas_call(
        functools.partial(_stage_body, float(n), br, relu, matmul),
        grid=(2, grid),
        in_specs=[
            pl.BlockSpec((NC, br, d), lambda ph, i: (0, i * (1 - ph), 0)),
            pl.BlockSpec((br, d), lambda ph, i: (i * (1 - ph), 0)),
            pl.BlockSpec((br, 2), lambda ph, i: (i, 0)),
            pl.BlockSpec((1, d), lambda ph, i: (0, 0)),
            pl.BlockSpec((1, d), lambda ph, i: (0, 0)),
            pl.BlockSpec((d, d), lambda ph, i: (0, 0)),
        ],
        out_specs=pl.BlockSpec((br, d), lambda ph, i: (i, 0)),
        out_shape=jax.ShapeDtypeStruct((n, d), F32),
        scratch_shapes=[
            pltpu.VMEM((n, d), F32),
            pltpu.VMEM((2, d), F32),
        ],
    )(part, hp, deg2, g.reshape(1, d), be.reshape(1, d), w)


def kernel(x, edge_index, edge_attr, batch, W0, b0, g0, be0, W1, b1, g1, be1):
    n, d = x.shape
    e = edge_index.shape[1]

    align = NW * CHUNK_D
    epad = -(-e // align) * align
    nchunk = epad // (NW * CHUNK)
    nchunk_d = epad // (NW * CHUNK_D)
    npad = (n // (NS * CHUNK) + 1) * (NS * CHUNK)
    br = 2000 if n % 2000 == 0 else 8 * (n // 8)

    row = edge_index[0].astype(jnp.int32)
    col = edge_index[1].astype(jnp.int32)
    nex = epad - e
    pad_src = (jnp.arange(nex, dtype=jnp.int32) * 97) % n
    pad_dst = n + jnp.arange(nex, dtype=jnp.int32) % (npad - n)
    col_f = jnp.concatenate([col, pad_dst])
    row_t = jnp.concatenate([row, pad_src]).reshape(NW, nchunk, CHUNK)
    col_t = col_f.reshape(NW, nchunk, CHUNK)
    col_d = col_f.reshape(NW, nchunk_d, CHUNK_D)

    deg_part = _make_deg_kernel(nchunk_d, npad)(col_d)
    deg2 = deg_part.T

    scatter = _make_scatter_kernel(nchunk, npad, d)

    h = x.astype(F32)
    hp0 = _hprime(h, W0, deg2, br)
    part0 = scatter(hp0, row_t, col_t)
    hp1 = _stage(part0, hp0, deg2, g0, be0, W1, br, relu=True, matmul=True)
    part1 = scatter(hp1, row_t, col_t)
    out = _stage(part1, hp1, deg2, g1, be1, W1, br, relu=False,
                 matmul=False)
    return out


# speedup vs baseline: 1.2352x; 1.0379x over previous
import functools

import jax
import jax.numpy as jnp
from jax import lax
from jax.experimental import pallas as pl
from jax.experimental.pallas import tpu as pltpu
from jax.experimental.pallas import tpu_sc as plsc

F32 = jnp.float32
L = 16
NC = 2
NS = 16
NW = NC * NS
CHUNK = 64
CHUNK_D = 128


def _sc_mesh():
    return plsc.VectorSubcoreMesh(core_axis_name="c", subcore_axis_name="s")


def _zero_vmem_2d(ref, rows, cols):
    def body(i, _):
        def inner(j, _):
            ref[i, pl.ds(j * L, L)] = jnp.zeros((L,), F32)
            return 0
        return lax.fori_loop(0, cols // L, inner, 0)
    lax.fori_loop(0, rows, body, 0)


def _make_deg_kernel(nchunk, npad):
    sl = npad // NS

    @functools.partial(
        pl.kernel,
        out_type=jax.ShapeDtypeStruct((NC, npad), F32),
        mesh=_sc_mesh(),
        scratch_types=[
            pltpu.VMEM((nchunk, CHUNK_D), jnp.int32),
            pltpu.VMEM((CHUNK_D,), F32),
            pltpu.VMEM((sl,), F32),
            pltpu.VMEM_SHARED((npad,), F32),
            [pltpu.SemaphoreType.DMA] * 4,
        ],
    )
    def deg_kernel(col_hbm, deg_out, col_v, ones_v, tmp_v, shared_deg, sems):
        c = lax.axis_index("c")
        s = lax.axis_index("s")
        wid = s * NC + c

        def zb(i, _):
            tmp_v[pl.ds(i * L, L)] = jnp.zeros((L,), F32)
            return 0
        lax.fori_loop(0, sl // L, zb, 0)

        def ob(i, _):
            ones_v[pl.ds(i * L, L)] = jnp.ones((L,), F32)
            return 0
        lax.fori_loop(0, CHUNK_D // L, ob, 0)

        pltpu.sync_copy(tmp_v, shared_deg.at[pl.ds(s * sl, sl)])
        plsc.subcore_barrier()

        pltpu.sync_copy(col_hbm.at[wid], col_v)

        def dscat(j, k):
            pltpu.async_copy(ones_v, shared_deg.at[col_v.at[j]], sems[k],
                             add=True)

        def dscat_wait(j, k):
            pltpu.make_async_copy(ones_v, shared_deg.at[col_v.at[j]],
                                  sems[k]).wait()

        for k in range(4):
            if k < nchunk:
                dscat(k, k)

        def edge_ring(i, _):
            j = i * 4
            for k in range(4):
                @pl.when(j + k < nchunk)
                def _(k=k):
                    dscat_wait(j + k, k)

                    @pl.when(j + k + 4 < nchunk)
                    def _():
                        dscat(j + k + 4, k)
            return 0
        lax.fori_loop(0, -(-nchunk // 4), edge_ring, 0)

        plsc.subcore_barrier()
        pltpu.sync_copy(shared_deg.at[pl.ds(s * sl, sl)], tmp_v)
        pltpu.sync_copy(tmp_v, deg_out.at[c, pl.ds(s * sl, sl)])

    return deg_kernel


NB = 32
DEPTH = 4


def _make_scatter_kernel(nchunk, npad, d):
    sl = npad // NS
    kch = sl // CHUNK
    nblk = -(-nchunk // NB)

    @functools.partial(
        pl.kernel,
        out_type=jax.ShapeDtypeStruct((NC, npad, d), F32),
        mesh=_sc_mesh(),
        scratch_types=[
            pltpu.VMEM((2, NB, CHUNK), jnp.int32),
            pltpu.VMEM((2, NB, CHUNK), jnp.int32),
            pltpu.VMEM((DEPTH, CHUNK, d), F32),
            pltpu.VMEM_SHARED((npad, d), F32),
            [pltpu.SemaphoreType.DMA] * DEPTH,
            pltpu.SemaphoreType.DMA,
        ],
    )
    def scatter_kernel(hp_hbm, row_hbm, col_hbm, part_out,
                       row_v, col_v, buf, acc, gsems, isem):
        c = lax.axis_index("c")
        s = lax.axis_index("s")
        wid = s * NC + c

        _zero_vmem_2d(buf.at[0], CHUNK, d)
        for k in range(kch):
            pltpu.sync_copy(buf.at[0],
                            acc.at[pl.ds(s * sl + k * CHUNK, CHUNK)])
        plsc.subcore_barrier()

        def idx_copies(b, nb):
            slot = b % 2
            src_r = row_hbm.at[wid, pl.ds(b * NB, nb)]
            src_c = col_hbm.at[wid, pl.ds(b * NB, nb)]
            dst_r = row_v.at[slot, pl.ds(0, nb)]
            dst_c = col_v.at[slot, pl.ds(0, nb)]
            return (src_r, dst_r), (src_c, dst_c)

        (sr, dr), (sc_, dc) = idx_copies(0, min(NB, nchunk))
        pltpu.sync_copy(sr, dr)
        pltpu.sync_copy(sc_, dc)

        for b in range(nblk):
            slot = b % 2
            nb = min(NB, nchunk - b * NB)
            if b > 0:
                (sr, dr), (sc_, dc) = idx_copies(b, nb)
                pltpu.make_async_copy(sr, dr, isem).wait()
                pltpu.make_async_copy(sc_, dc, isem).wait()
            if b + 1 < nblk:
                (sr, dr), (sc_, dc) = idx_copies(b + 1,
                                                 min(NB, nchunk - (b + 1) * NB))
                pltpu.async_copy(sr, dr, isem)
                pltpu.async_copy(sc_, dc, isem)

            rv = row_v.at[slot]
            cv = col_v.at[slot]

            def gath(j, k):
                pltpu.async_copy(hp_hbm.at[rv.at[j]], buf.at[k], gsems[k])

            def gath_wait(j, k):
                pltpu.make_async_copy(hp_hbm.at[rv.at[j]], buf.at[k],
                                      gsems[k]).wait()

            for k in range(DEPTH):
                if k < nb:
                    gath(k, k)

            def quad(i, _):
                j = i * DEPTH
                for k in range(DEPTH):
                    @pl.when(j + k < nb)
                    def _(k=k):
                        gath_wait(j + k, k)
                        pltpu.sync_copy(buf.at[k], acc.at[cv.at[j + k]],
                                        add=True)

                        @pl.when(j + k + DEPTH < nb)
                        def _():
                            gath(j + k + DEPTH, k)
                return 0
            lax.fori_loop(0, -(-nb // DEPTH), quad, 0)

        plsc.subcore_barrier()
        def out_desc(k):
            base = s * sl + k * CHUNK
            return (buf.at[k % 2], part_out.at[c, pl.ds(base, CHUNK)],
                    gsems[k % 2])

        for k in range(kch):
            if k >= 2:
                pltpu.make_async_copy(*out_desc(k - 2)).wait()
            pltpu.sync_copy(acc.at[pl.ds(s * sl + k * CHUNK, CHUNK)],
                            buf.at[k % 2])
            pltpu.async_copy(*out_desc(k))
        for k in range(max(kch - 2, 0), kch):
            pltpu.make_async_copy(*out_desc(k)).wait()

    return scatter_kernel


def _dinv_of(deg2_blk):
    return lax.rsqrt(deg2_blk[:, 0] + deg2_blk[:, 1] + 1.0)


def _hprime_body(x_ref, w_ref, deg2_ref, out_ref):
    dinv = _dinv_of(deg2_ref[...])
    h = jnp.dot(x_ref[...], w_ref[...], preferred_element_type=F32)
    out_ref[...] = h * dinv[:, None]


def _hprime(x, w, deg2, br):
    n, d = x.shape
    grid = n // br
    return pl.pallas_call(
        _hprime_body,
        grid=(grid,),
        in_specs=[
            pl.BlockSpec((br, d), lambda i: (i, 0)),
            pl.BlockSpec((d, d), lambda i: (0, 0)),
            pl.BlockSpec((br, 2), lambda i: (i, 0)),
        ],
        out_specs=pl.BlockSpec((br, d), lambda i: (i, 0)),
        out_shape=jax.ShapeDtypeStruct((n, d), F32),
    )(x, w, deg2)


def _stage_body(n, br, relu, matmul, part_ref, hp_ref, deg2_ref, g_ref,
                be_ref, w_ref, out_ref, c_s, sums_s):
    ph = pl.program_id(0)
    i = pl.program_id(1)
    dinv = _dinv_of(deg2_ref[...])

    @pl.when(ph == 0)
    def _():
        cc = (part_ref[0] + part_ref[1] + hp_ref[...]) * dinv[:, None]
        c_s[pl.ds(i * br, br), :] = cc

        @pl.when(i == 0)
        def _():
            sums_s[...] = jnp.zeros_like(sums_s)
        sums_s[...] += jnp.stack([cc.sum(axis=0), (cc * cc).sum(axis=0)])

    @pl.when(ph == 1)
    def _():
        mu = sums_s[0] / n
        var = sums_s[1] / n - mu * mu
        istd = lax.rsqrt(var + 1e-5)
        y = (c_s[pl.ds(i * br, br), :] - mu) * (istd * g_ref[0]) + be_ref[0]
        if relu:
            y = jnp.maximum(y, 0.0)
        if matmul:
            y = (jnp.dot(y, w_ref[...], preferred_element_type=F32)
                 * dinv[:, None])
        out_ref[...] = y


def _stage(part, hp, deg2, g, be, w, br, relu, matmul):
    n, d = hp.shape
    grid = n // br
    return pl.pallas_call(
        functools.partial(_stage_body, float(n), br, relu, matmul),
        grid=(2, grid),
        in_specs=[
            pl.BlockSpec((NC, br, d), lambda ph, i: (0, i * (1 - ph), 0)),
            pl.BlockSpec((br, d), lambda ph, i: (i * (1 - ph), 0)),
            pl.BlockSpec((br, 2), lambda ph, i: (i, 0)),
            pl.BlockSpec((1, d), lambda ph, i: (0, 0)),
            pl.BlockSpec((1, d), lambda ph, i: (0, 0)),
            pl.BlockSpec((d, d), lambda ph, i: (0, 0)),
        ],
        out_specs=pl.BlockSpec((br, d), lambda ph, i: (i, 0)),
        out_shape=jax.ShapeDtypeStruct((n, d), F32),
        scratch_shapes=[
            pltpu.VMEM((n, d), F32),
            pltpu.VMEM((2, d), F32),
        ],
    )(part, hp, deg2, g.reshape(1, d), be.reshape(1, d), w)


def kernel(x, edge_index, edge_attr, batch, W0, b0, g0, be0, W1, b1, g1, be1):
    n, d = x.shape
    e = edge_index.shape[1]

    align = NW * CHUNK_D
    epad = -(-e // align) * align
    nchunk = epad // (NW * CHUNK)
    nchunk_d = epad // (NW * CHUNK_D)
    npad = (n // (NS * CHUNK) + 1) * (NS * CHUNK)
    br = 2000 if n % 2000 == 0 else 8 * (n // 8)

    row = edge_index[0].astype(jnp.int32)
    col = edge_index[1].astype(jnp.int32)
    nex = epad - e
    pad_src = (jnp.arange(nex, dtype=jnp.int32) * 97) % n
    pad_dst = n + jnp.arange(nex, dtype=jnp.int32) % (npad - n)
    col_f = jnp.concatenate([col, pad_dst])
    row_t = jnp.concatenate([row, pad_src]).reshape(NW, nchunk, CHUNK)
    col_t = col_f.reshape(NW, nchunk, CHUNK)
    col_d = col_f.reshape(NW, nchunk_d, CHUNK_D)

    deg_part = _make_deg_kernel(nchunk_d, npad)(col_d)
    deg2 = deg_part.T

    scatter = _make_scatter_kernel(nchunk, npad, d)

    h = x.astype(F32)
    hp0 = _hprime(h, W0, deg2, br)
    part0 = scatter(hp0, row_t, col_t)
    hp1 = _stage(part0, hp0, deg2, g0, be0, W1, br, relu=True, matmul=True)
    part1 = scatter(hp1, row_t, col_t)
    out = _stage(part1, hp1, deg2, g1, be1, W1, br, relu=False,
                 matmul=False)
    return out


# speedup vs baseline: 1.2528x; 1.0142x over previous
import functools

import jax
import jax.numpy as jnp
from jax import lax
from jax.experimental import pallas as pl
from jax.experimental.pallas import tpu as pltpu
from jax.experimental.pallas import tpu_sc as plsc

F32 = jnp.float32
L = 16
NC = 2
NS = 16
NW = NC * NS
CHUNK = 64
CHUNK_D = 128


def _sc_mesh():
    return plsc.VectorSubcoreMesh(core_axis_name="c", subcore_axis_name="s")


def _zero_vmem_2d(ref, rows, cols):
    def body(i, _):
        def inner(j, _):
            ref[i, pl.ds(j * L, L)] = jnp.zeros((L,), F32)
            return 0
        return lax.fori_loop(0, cols // L, inner, 0)
    lax.fori_loop(0, rows, body, 0)


def _make_deg_kernel(nchunk, npad):
    sl = npad // NS

    @functools.partial(
        pl.kernel,
        out_type=jax.ShapeDtypeStruct((NC, npad), F32),
        mesh=_sc_mesh(),
        scratch_types=[
            pltpu.VMEM((nchunk, CHUNK_D), jnp.int32),
            pltpu.VMEM((CHUNK_D,), F32),
            pltpu.VMEM((sl,), F32),
            pltpu.VMEM_SHARED((npad,), F32),
            [pltpu.SemaphoreType.DMA] * 4,
        ],
    )
    def deg_kernel(col_hbm, deg_out, col_v, ones_v, tmp_v, shared_deg, sems):
        c = lax.axis_index("c")
        s = lax.axis_index("s")
        wid = s * NC + c

        def zb(i, _):
            tmp_v[pl.ds(i * L, L)] = jnp.zeros((L,), F32)
            return 0
        lax.fori_loop(0, sl // L, zb, 0)

        def ob(i, _):
            ones_v[pl.ds(i * L, L)] = jnp.ones((L,), F32)
            return 0
        lax.fori_loop(0, CHUNK_D // L, ob, 0)

        pltpu.sync_copy(tmp_v, shared_deg.at[pl.ds(s * sl, sl)])
        plsc.subcore_barrier()

        pltpu.sync_copy(col_hbm.at[wid], col_v)

        def dscat(j, k):
            pltpu.async_copy(ones_v, shared_deg.at[col_v.at[j]], sems[k],
                             add=True)

        def dscat_wait(j, k):
            pltpu.make_async_copy(ones_v, shared_deg.at[col_v.at[j]],
                                  sems[k]).wait()

        for k in range(4):
            if k < nchunk:
                dscat(k, k)

        def edge_ring(i, _):
            j = i * 4
            for k in range(4):
                @pl.when(j + k < nchunk)
                def _(k=k):
                    dscat_wait(j + k, k)

                    @pl.when(j + k + 4 < nchunk)
                    def _():
                        dscat(j + k + 4, k)
            return 0
        lax.fori_loop(0, -(-nchunk // 4), edge_ring, 0)

        plsc.subcore_barrier()
        pltpu.sync_copy(shared_deg.at[pl.ds(s * sl, sl)], tmp_v)
        pltpu.sync_copy(tmp_v, deg_out.at[c, pl.ds(s * sl, sl)])

    return deg_kernel


NB = 32
DEPTH = 4


def _make_scatter_kernel(nchunk, npad, d):
    sl = npad // NS
    kch = sl // CHUNK
    nblk = -(-nchunk // NB)

    @functools.partial(
        pl.kernel,
        out_type=jax.ShapeDtypeStruct((NC, npad, d), F32),
        mesh=_sc_mesh(),
        scratch_types=[
            pltpu.VMEM((2, NB, CHUNK), jnp.int32),
            pltpu.VMEM((2, NB, CHUNK), jnp.int32),
            pltpu.VMEM((DEPTH, CHUNK, d), F32),
            pltpu.VMEM_SHARED((npad, d), F32),
            [pltpu.SemaphoreType.DMA] * DEPTH,
            pltpu.SemaphoreType.DMA,
        ],
    )
    def scatter_kernel(hp_hbm, row_hbm, col_hbm, part_out,
                       row_v, col_v, buf, acc, gsems, isem):
        c = lax.axis_index("c")
        s = lax.axis_index("s")
        wid = s * NC + c

        def idx_copies(b, nb):
            slot = b % 2
            src_r = row_hbm.at[wid, pl.ds(b * NB, nb)]
            src_c = col_hbm.at[wid, pl.ds(b * NB, nb)]
            dst_r = row_v.at[slot, pl.ds(0, nb)]
            dst_c = col_v.at[slot, pl.ds(0, nb)]
            return (src_r, dst_r), (src_c, dst_c)

        (sr, dr), (sc_, dc) = idx_copies(0, min(NB, nchunk))
        pltpu.async_copy(sr, dr, isem)
        pltpu.async_copy(sc_, dc, isem)

        _zero_vmem_2d(buf.at[0], CHUNK, d)

        def zdesc(k):
            base = s * sl + k * CHUNK
            return (buf.at[0], acc.at[pl.ds(base, CHUNK)], gsems[k % 4])

        for k in range(kch):
            if k >= 4:
                pltpu.make_async_copy(*zdesc(k - 4)).wait()
            pltpu.async_copy(*zdesc(k))
        for k in range(max(kch - 4, 0), kch):
            pltpu.make_async_copy(*zdesc(k)).wait()
        plsc.subcore_barrier()

        pltpu.make_async_copy(sr, dr, isem).wait()
        pltpu.make_async_copy(sc_, dc, isem).wait()

        for b in range(nblk):
            slot = b % 2
            nb = min(NB, nchunk - b * NB)
            if b > 0:
                (sr, dr), (sc_, dc) = idx_copies(b, nb)
                pltpu.make_async_copy(sr, dr, isem).wait()
                pltpu.make_async_copy(sc_, dc, isem).wait()
            if b + 1 < nblk:
                (sr, dr), (sc_, dc) = idx_copies(b + 1,
                                                 min(NB, nchunk - (b + 1) * NB))
                pltpu.async_copy(sr, dr, isem)
                pltpu.async_copy(sc_, dc, isem)

            rv = row_v.at[slot]
            cv = col_v.at[slot]

            def gath(j, k):
                pltpu.async_copy(hp_hbm.at[rv.at[j]], buf.at[k], gsems[k])

            def gath_wait(j, k):
                pltpu.make_async_copy(hp_hbm.at[rv.at[j]], buf.at[k],
                                      gsems[k]).wait()

            for k in range(DEPTH):
                if k < nb:
                    gath(k, k)

            def quad(i, _):
                j = i * DEPTH
                for k in range(DEPTH):
                    @pl.when(j + k < nb)
                    def _(k=k):
                        gath_wait(j + k, k)
                        pltpu.sync_copy(buf.at[k], acc.at[cv.at[j + k]],
                                        add=True)

                        @pl.when(j + k + DEPTH < nb)
                        def _():
                            gath(j + k + DEPTH, k)
                return 0
            lax.fori_loop(0, -(-nb // DEPTH), quad, 0)

        plsc.subcore_barrier()
        def out_desc(k):
            base = s * sl + k * CHUNK
            return (buf.at[k % 4], part_out.at[c, pl.ds(base, CHUNK)],
                    gsems[k % 4])

        for k in range(kch):
            if k >= 4:
                pltpu.make_async_copy(*out_desc(k - 4)).wait()
            pltpu.sync_copy(acc.at[pl.ds(s * sl + k * CHUNK, CHUNK)],
                            buf.at[k % 4])
            pltpu.async_copy(*out_desc(k))
        for k in range(max(kch - 4, 0), kch):
            pltpu.make_async_copy(*out_desc(k)).wait()

    return scatter_kernel


def _dinv_of(deg2_blk):
    return lax.rsqrt(deg2_blk[:, 0] + deg2_blk[:, 1] + 1.0)


def _hprime_body(x_ref, w_ref, deg2_ref, out_ref):
    dinv = _dinv_of(deg2_ref[...])
    h = jnp.dot(x_ref[...], w_ref[...], preferred_element_type=F32)
    out_ref[...] = h * dinv[:, None]


def _hprime(x, w, deg2, br):
    n, d = x.shape
    grid = n // br
    return pl.pallas_call(
        _hprime_body,
        grid=(grid,),
        in_specs=[
            pl.BlockSpec((br, d), lambda i: (i, 0)),
            pl.BlockSpec((d, d), lambda i: (0, 0)),
            pl.BlockSpec((br, 2), lambda i: (i, 0)),
        ],
        out_specs=pl.BlockSpec((br, d), lambda i: (i, 0)),
        out_shape=jax.ShapeDtypeStruct((n, d), F32),
    )(x, w, deg2)


def _stage_body(n, br, relu, matmul, part_ref, hp_ref, deg2_ref, g_ref,
                be_ref, w_ref, out_ref, c_s, sums_s):
    ph = pl.program_id(0)
    i = pl.program_id(1)
    dinv = _dinv_of(deg2_ref[...])

    @pl.when(ph == 0)
    def _():
        cc = (part_ref[0] + part_ref[1] + hp_ref[...]) * dinv[:, None]
        c_s[pl.ds(i * br, br), :] = cc

        @pl.when(i == 0)
        def _():
            sums_s[...] = jnp.zeros_like(sums_s)
        sums_s[...] += jnp.stack([cc.sum(axis=0), (cc * cc).sum(axis=0)])

    @pl.when(ph == 1)
    def _():
        mu = sums_s[0] / n
        var = sums_s[1] / n - mu * mu
        istd = lax.rsqrt(var + 1e-5)
        y = (c_s[pl.ds(i * br, br), :] - mu) * (istd * g_ref[0]) + be_ref[0]
        if relu:
            y = jnp.maximum(y, 0.0)
        if matmul:
            y = (jnp.dot(y, w_ref[...], preferred_element_type=F32)
                 * dinv[:, None])
        out_ref[...] = y


def _stage(part, hp, deg2, g, be, w, br, relu, matmul):
    n, d = hp.shape
    grid = n // br
    return pl.pallas_call(
        functools.partial(_stage_body, float(n), br, relu, matmul),
        grid=(2, grid),
        in_specs=[
            pl.BlockSpec((NC, br, d), lambda ph, i: (0, i * (1 - ph), 0)),
            pl.BlockSpec((br, d), lambda ph, i: (i * (1 - ph), 0)),
            pl.BlockSpec((br, 2), lambda ph, i: (i, 0)),
            pl.BlockSpec((1, d), lambda ph, i: (0, 0)),
            pl.BlockSpec((1, d), lambda ph, i: (0, 0)),
            pl.BlockSpec((d, d), lambda ph, i: (0, 0)),
        ],
        out_specs=pl.BlockSpec((br, d), lambda ph, i: (i, 0)),
        out_shape=jax.ShapeDtypeStruct((n, d), F32),
        scratch_shapes=[
            pltpu.VMEM((n, d), F32),
            pltpu.VMEM((2, d), F32),
        ],
    )(part, hp, deg2, g.reshape(1, d), be.reshape(1, d), w)


def kernel(x, edge_index, edge_attr, batch, W0, b0, g0, be0, W1, b1, g1, be1):
    n, d = x.shape
    e = edge_index.shape[1]

    align = NW * CHUNK_D
    epad = -(-e // align) * align
    nchunk = epad // (NW * CHUNK)
    nchunk_d = epad // (NW * CHUNK_D)
    npad = (n // (NS * CHUNK) + 1) * (NS * CHUNK)
    br = 2000 if n % 2000 == 0 else 8 * (n // 8)

    row = edge_index[0].astype(jnp.int32)
    col = edge_index[1].astype(jnp.int32)
    nex = epad - e
    pad_src = (jnp.arange(nex, dtype=jnp.int32) * 97) % n
    pad_dst = n + jnp.arange(nex, dtype=jnp.int32) % (npad - n)
    col_f = jnp.concatenate([col, pad_dst])
    row_t = jnp.concatenate([row, pad_src]).reshape(NW, nchunk, CHUNK)
    col_t = col_f.reshape(NW, nchunk, CHUNK)
    col_d = col_f.reshape(NW, nchunk_d, CHUNK_D)

    deg_part = _make_deg_kernel(nchunk_d, npad)(col_d)
    deg2 = deg_part.T

    scatter = _make_scatter_kernel(nchunk, npad, d)

    h = x.astype(F32)
    hp0 = _hprime(h, W0, deg2, br)
    part0 = scatter(hp0, row_t, col_t)
    hp1 = _stage(part0, hp0, deg2, g0, be0, W1, br, relu=True, matmul=True)
    part1 = scatter(hp1, row_t, col_t)
    out = _stage(part1, hp1, deg2, g1, be1, W1, br, relu=False,
                 matmul=False)
    return out
